# Initial kernel scaffold; baseline (speedup 1.0000x reference)
import jax
import jax.numpy as jnp
from jax.experimental import pallas as pl


def kernel(X, Z):
    raise NotImplementedError("write your pallas kernel here")


# speedup vs baseline: 10.9107x; 10.9107x over previous
import jax
import jax.numpy as jnp
from jax.experimental import pallas as pl
from jax.experimental.pallas import tpu as pltpu

_K = 8
_EPS = 1e-07
_BLK = 256


def _lid_nsa_kernel(x_blk, xt, z_blk, zt, out, *, n, blk, k):
    b = pl.program_id(0)

    xsq_cols = jnp.sum(xt[...] * xt[...], axis=0, keepdims=True)
    zsq_cols = jnp.sum(zt[...] * zt[...], axis=0, keepdims=True)
    xsq_rows = jnp.sum(x_blk[...] * x_blk[...], axis=1, keepdims=True)
    zsq_rows = jnp.sum(z_blk[...] * z_blk[...], axis=1, keepdims=True)

    dots_x = jax.lax.dot_general(
        x_blk[...], xt[...], (((1,), (0,)), ((), ())),
        preferred_element_type=jnp.float32,
        precision=jax.lax.Precision.DEFAULT,
    )
    dots_z = jax.lax.dot_general(
        z_blk[...], zt[...], (((1,), (0,)), ((), ())),
        preferred_element_type=jnp.float32,
        precision=jax.lax.Precision.DEFAULT,
    )

    d2x = xsq_rows + xsq_cols - 2.0 * dots_x
    d2z = zsq_rows + zsq_cols - 2.0 * dots_z
    dz = jnp.sqrt(jnp.maximum(d2z, 1e-12)) + _EPS

    col_iota = jax.lax.broadcasted_iota(jnp.int32, (blk, n), 1)
    row_iota = jax.lax.broadcasted_iota(jnp.int32, (blk, n), 0)
    diag = col_iota == (row_iota + b * blk)

    dx = jnp.sqrt(jnp.maximum(d2x, 1e-12))
    dx = jnp.where(diag, jnp.inf, dx)

    sum_log_x = jnp.zeros((blk, 1), jnp.float32)
    sum_log_z = jnp.zeros((blk, 1), jnp.float32)
    max_z = jnp.zeros((blk, 1), jnp.float32)
    last_x = jnp.zeros((blk, 1), jnp.float32)
    for _ in range(k):
        m = jnp.min(dx, axis=1, keepdims=True)
        idx = jnp.min(jnp.where(dx == m, col_iota, n), axis=1, keepdims=True)
        onehot = col_iota == idx
        zval = jnp.sum(jnp.where(onehot, dz, 0.0), axis=1, keepdims=True)
        sum_log_x = sum_log_x + jnp.log10(m + _EPS)
        sum_log_z = sum_log_z + jnp.log10(zval)
        max_z = jnp.maximum(max_z, zval)
        last_x = m
        dx = jnp.where(onehot, jnp.inf, dx)

    lid_x = sum_log_x - k * jnp.log10(last_x + _EPS)
    lid_z = sum_log_z - k * jnp.log10(max_z)
    diff = k / (lid_x + _EPS) - k / (lid_z + _EPS)
    partial = jnp.sum(diff * diff, keepdims=True).reshape(1, 1)

    @pl.when(b == 0)
    def _():
        out[...] = jnp.zeros_like(out)

    out[...] += partial


def kernel(X, Z):
    n, _ = X.shape
    XT = X.T
    ZT = Z.T
    grid = n // _BLK

    import functools
    body = functools.partial(_lid_nsa_kernel, n=n, blk=_BLK, k=_K)

    out = pl.pallas_call(
        body,
        grid=(grid,),
        in_specs=[
            pl.BlockSpec((_BLK, X.shape[1]), lambda b: (b, 0)),
            pl.BlockSpec((X.shape[1], n), lambda b: (0, 0)),
            pl.BlockSpec((_BLK, Z.shape[1]), lambda b: (b, 0)),
            pl.BlockSpec((Z.shape[1], n), lambda b: (0, 0)),
        ],
        out_specs=pl.BlockSpec((1, 1), lambda b: (0, 0)),
        out_shape=jax.ShapeDtypeStruct((1, 1), jnp.float32),
    )(X, XT, Z, ZT)

    return (out[0, 0] / (n * _K * _K)).astype(jnp.float32)


# speedup vs baseline: 17.4117x; 1.5958x over previous
import functools

import jax
import jax.numpy as jnp
from jax.experimental import pallas as pl
from jax.experimental.pallas import tpu as pltpu

_K = 8
_EPS = 1e-07
_BLK = 256


def _lid_nsa_kernel(x_blk, xt, z_blk, zt, out, xc_ref, zc_ref, dio_ref,
                    *, n, blk, k):
    b = pl.program_id(0)

    @pl.when(b == 0)
    def _():
        xc_ref[...] = jnp.sum(xt[...] * xt[...], axis=0, keepdims=True)
        zc_ref[...] = jnp.sum(zt[...] * zt[...], axis=0, keepdims=True)
        col_i = jax.lax.broadcasted_iota(jnp.int32, (blk, n), 1)
        row_i = jax.lax.broadcasted_iota(jnp.int32, (blk, n), 0)
        dio_ref[...] = (col_i - row_i).astype(jnp.float32)

    xsq_rows = jnp.sum(x_blk[...] * x_blk[...], axis=1, keepdims=True)
    zsq_rows = jnp.sum(z_blk[...] * z_blk[...], axis=1, keepdims=True)

    dots_x = jax.lax.dot_general(
        x_blk[...], xt[...], (((1,), (0,)), ((), ())),
        preferred_element_type=jnp.float32,
        precision=jax.lax.Precision.DEFAULT,
    )
    dots_z = jax.lax.dot_general(
        z_blk[...], zt[...], (((1,), (0,)), ((), ())),
        preferred_element_type=jnp.float32,
        precision=jax.lax.Precision.DEFAULT,
    )

    tx = xc_ref[...] - 2.0 * dots_x
    tz = zc_ref[...] - 2.0 * dots_z
    tx = jnp.where(dio_ref[...] == (b * blk).astype(jnp.float32), jnp.inf, tx)

    sum_log_x = jnp.zeros((blk, 1), jnp.float32)
    sum_log_z = jnp.zeros((blk, 1), jnp.float32)
    max_z = jnp.zeros((blk, 1), jnp.float32)
    last_x = jnp.zeros((blk, 1), jnp.float32)
    neg_inf = jnp.float32(-jnp.inf)
    for _ in range(k):
        m = jnp.min(tx, axis=1, keepdims=True)
        eq = tx == m
        zext = jnp.max(jnp.where(eq, tz, neg_inf), axis=1, keepdims=True)
        tx = jnp.where(eq, jnp.inf, tx)
        d2x = jnp.maximum(xsq_rows + m, 1e-12)
        d2z = jnp.maximum(zsq_rows + zext, 1e-12)
        vx = jnp.sqrt(d2x) + _EPS
        vz = jnp.sqrt(d2z) + _EPS
        sum_log_x = sum_log_x + jnp.log10(vx)
        sum_log_z = sum_log_z + jnp.log10(vz)
        max_z = jnp.maximum(max_z, vz)
        last_x = vx

    lid_x = sum_log_x - k * jnp.log10(last_x)
    lid_z = sum_log_z - k * jnp.log10(max_z)
    diff = k / (lid_x + _EPS) - k / (lid_z + _EPS)
    partial = jnp.sum(diff * diff, keepdims=True).reshape(1, 1)

    @pl.when(b == 0)
    def _():
        out[...] = jnp.zeros_like(out)

    out[...] += partial


def kernel(X, Z):
    n, dx = X.shape
    _, dz = Z.shape
    XT = X.T
    ZT = Z.T
    grid = n // _BLK

    body = functools.partial(_lid_nsa_kernel, n=n, blk=_BLK, k=_K)

    out = pl.pallas_call(
        body,
        grid=(grid,),
        in_specs=[
            pl.BlockSpec((_BLK, dx), lambda b: (b, 0)),
            pl.BlockSpec((dx, n), lambda b: (0, 0)),
            pl.BlockSpec((_BLK, dz), lambda b: (b, 0)),
            pl.BlockSpec((dz, n), lambda b: (0, 0)),
        ],
        out_specs=pl.BlockSpec((1, 1), lambda b: (0, 0)),
        out_shape=jax.ShapeDtypeStruct((1, 1), jnp.float32),
        scratch_shapes=[
            pltpu.VMEM((1, n), jnp.float32),
            pltpu.VMEM((1, n), jnp.float32),
            pltpu.VMEM((_BLK, n), jnp.float32),
        ],
    )(X, XT, Z, ZT)

    return (out[0, 0] / (n * _K * _K)).astype(jnp.float32)


# speedup vs baseline: 17.4347x; 1.0013x over previous
import functools

import jax
import jax.numpy as jnp
from jax.experimental import pallas as pl
from jax.experimental.pallas import tpu as pltpu

_K = 8
_EPS = 1e-07
_BLK = 256


def _lid_nsa_kernel(x_blk, xt, z_blk, zt, out, xc_ref, zc_ref, dio_ref,
                    *, n, blk, k):
    b = pl.program_id(0)

    @pl.when(b == 0)
    def _():
        xc_ref[...] = jnp.sum(xt[...] * xt[...], axis=0, keepdims=True)
        zc_ref[...] = jnp.sum(zt[...] * zt[...], axis=0, keepdims=True)
        col_i = jax.lax.broadcasted_iota(jnp.int32, (blk, n), 1)
        row_i = jax.lax.broadcasted_iota(jnp.int32, (blk, n), 0)
        dio_ref[...] = (col_i - row_i).astype(jnp.float32)

    xsq_rows = jnp.sum(x_blk[...] * x_blk[...], axis=1, keepdims=True)
    zsq_rows = jnp.sum(z_blk[...] * z_blk[...], axis=1, keepdims=True)

    dots2_x = jax.lax.dot_general(
        x_blk[...] * 2.0, xt[...], (((1,), (0,)), ((), ())),
        preferred_element_type=jnp.float32,
        precision=jax.lax.Precision.DEFAULT,
    )
    dots2_z = jax.lax.dot_general(
        z_blk[...] * 2.0, zt[...], (((1,), (0,)), ((), ())),
        preferred_element_type=jnp.float32,
        precision=jax.lax.Precision.DEFAULT,
    )

    tx = xc_ref[...] - dots2_x
    tz = zc_ref[...] - dots2_z
    tx = jnp.where(dio_ref[...] == (b * blk).astype(jnp.float32), jnp.inf, tx)

    sum_log_x = jnp.zeros((blk, 1), jnp.float32)
    sum_log_z = jnp.zeros((blk, 1), jnp.float32)
    max_z = jnp.zeros((blk, 1), jnp.float32)
    last_x = jnp.zeros((blk, 1), jnp.float32)
    neg_inf = jnp.float32(-jnp.inf)
    for _ in range(k):
        m = jnp.min(tx, axis=1, keepdims=True)
        eq = tx == m
        zext = jnp.max(jnp.where(eq, tz, neg_inf), axis=1, keepdims=True)
        tx = jnp.where(eq, jnp.inf, tx)
        d2x = jnp.maximum(xsq_rows + m, 1e-12)
        d2z = jnp.maximum(zsq_rows + zext, 1e-12)
        vx = jnp.sqrt(d2x) + _EPS
        vz = jnp.sqrt(d2z) + _EPS
        sum_log_x = sum_log_x + jnp.log10(vx)
        sum_log_z = sum_log_z + jnp.log10(vz)
        max_z = jnp.maximum(max_z, vz)
        last_x = vx

    lid_x = sum_log_x - k * jnp.log10(last_x)
    lid_z = sum_log_z - k * jnp.log10(max_z)
    diff = k / (lid_x + _EPS) - k / (lid_z + _EPS)
    partial = jnp.sum(diff * diff, keepdims=True).reshape(1, 1)

    @pl.when(b == 0)
    def _():
        out[...] = jnp.zeros_like(out)

    out[...] += partial


def kernel(X, Z):
    n, dx = X.shape
    _, dz = Z.shape
    XT = X.T
    ZT = Z.T
    grid = n // _BLK

    body = functools.partial(_lid_nsa_kernel, n=n, blk=_BLK, k=_K)

    out = pl.pallas_call(
        body,
        grid=(grid,),
        in_specs=[
            pl.BlockSpec((_BLK, dx), lambda b: (b, 0)),
            pl.BlockSpec((dx, n), lambda b: (0, 0)),
            pl.BlockSpec((_BLK, dz), lambda b: (b, 0)),
            pl.BlockSpec((dz, n), lambda b: (0, 0)),
        ],
        out_specs=pl.BlockSpec((1, 1), lambda b: (0, 0)),
        out_shape=jax.ShapeDtypeStruct((1, 1), jnp.float32),
        scratch_shapes=[
            pltpu.VMEM((1, n), jnp.float32),
            pltpu.VMEM((1, n), jnp.float32),
            pltpu.VMEM((_BLK, n), jnp.float32),
        ],
    )(X, XT, Z, ZT)

    return (out[0, 0] / (n * _K * _K)).astype(jnp.float32)


# speedup vs baseline: 24.0985x; 1.3822x over previous
import functools

import jax
import jax.numpy as jnp
from jax.experimental import pallas as pl
from jax.experimental.pallas import tpu as pltpu

_K = 8
_EPS = 1e-07
_BLK = 256


def _lid_nsa_kernel(x_blk, xt, z_blk, zt, out, xc_ref, zc_ref, dio_ref,
                    *, n, blk, k):
    b = pl.program_id(0)

    @pl.when(b == 0)
    def _():
        xc_ref[...] = jnp.sum(xt[...] * xt[...], axis=0, keepdims=True)
        zc_ref[...] = jnp.sum(zt[...] * zt[...], axis=0, keepdims=True)
        col_i = jax.lax.broadcasted_iota(jnp.int32, (blk, n), 1)
        row_i = jax.lax.broadcasted_iota(jnp.int32, (blk, n), 0)
        dio_ref[...] = (col_i - row_i).astype(jnp.float32)

    xsq_rows = jnp.sum(x_blk[...] * x_blk[...], axis=1, keepdims=True)
    zsq_rows = jnp.sum(z_blk[...] * z_blk[...], axis=1, keepdims=True)

    dots2_x = jax.lax.dot_general(
        x_blk[...] * 2.0, xt[...], (((1,), (0,)), ((), ())),
        preferred_element_type=jnp.float32,
        precision=jax.lax.Precision.DEFAULT,
    )
    dots2_z = jax.lax.dot_general(
        z_blk[...] * 2.0, zt[...], (((1,), (0,)), ((), ())),
        preferred_element_type=jnp.float32,
        precision=jax.lax.Precision.DEFAULT,
    )

    tx = xc_ref[...] - dots2_x
    tz = zc_ref[...] - dots2_z
    tx = jnp.where(dio_ref[...] == (b * blk).astype(jnp.float32), jnp.inf, tx)

    lanes = 128

    def _ce(r, i, j):
        a, b_ = r[i], r[j]
        r[i] = jnp.minimum(a, b_)
        r[j] = jnp.maximum(a, b_)

    def _sort8(r):
        for i, j in ((0, 1), (2, 3), (4, 5), (6, 7),
                     (0, 2), (1, 3), (1, 2),
                     (4, 6), (5, 7), (5, 6),
                     (0, 4), (1, 5), (2, 6), (3, 7),
                     (2, 4), (3, 5), (1, 2), (3, 4), (5, 6)):
            _ce(r, i, j)

    def _merge_low8(p, q, sort_out):
        low = [jnp.minimum(p[i], q[7 - i]) for i in range(8)]
        if sort_out:
            for i, j in ((0, 4), (1, 5), (2, 6), (3, 7),
                         (0, 2), (1, 3), (4, 6), (5, 7),
                         (0, 1), (2, 3), (4, 5), (6, 7)):
                _ce(low, i, j)
        return low

    groups = []
    for g in range(n // lanes // 8):
        r = [tx[:, (8 * g + c) * lanes:(8 * g + c + 1) * lanes]
             for c in range(8)]
        _sort8(r)
        groups.append(r)
    while len(groups) > 2:
        groups = [_merge_low8(groups[2 * i], groups[2 * i + 1], True)
                  for i in range(len(groups) // 2)]
    regs = _merge_low8(groups[0], groups[1], False)
    cand = jnp.concatenate(regs, axis=1)

    sum_log_x = jnp.zeros((blk, 1), jnp.float32)
    last_x = jnp.zeros((blk, 1), jnp.float32)
    m = jnp.zeros((blk, 1), jnp.float32)
    for _ in range(k):
        m = jnp.min(cand, axis=1, keepdims=True)
        cand = jnp.where(cand == m, jnp.inf, cand)
        d2x = jnp.maximum(xsq_rows + m, 1e-12)
        vx = jnp.sqrt(d2x) + _EPS
        sum_log_x = sum_log_x + jnp.log10(vx)
        last_x = vx

    mask = tx <= m
    d2z = jnp.maximum(zsq_rows + tz, 1e-12)
    pz = jnp.where(mask, d2z, 1.0)
    w = n
    while w > 128:
        w //= 2
        pz = pz[:, :w] * pz[:, w:2 * w]
    for s in (64, 32, 16, 8, 4, 2, 1):
        pz = pz * jnp.roll(pz, s, axis=1)
    prod_z = pz[:, :1]
    max_d2z = jnp.max(jnp.where(mask, d2z, 0.0), axis=1, keepdims=True)

    lid_x = sum_log_x - k * jnp.log10(last_x)
    lid_z = 0.5 * jnp.log10(prod_z) - (0.5 * k) * jnp.log10(max_d2z)
    diff = k / (lid_x + _EPS) - k / (lid_z + _EPS)
    partial = jnp.sum(diff * diff, keepdims=True).reshape(1, 1)

    @pl.when(b == 0)
    def _():
        out[...] = jnp.zeros_like(out)

    out[...] += partial


def kernel(X, Z):
    n, dx = X.shape
    _, dz = Z.shape
    XT = X.T
    ZT = Z.T
    grid = n // _BLK

    body = functools.partial(_lid_nsa_kernel, n=n, blk=_BLK, k=_K)

    out = pl.pallas_call(
        body,
        grid=(grid,),
        in_specs=[
            pl.BlockSpec((_BLK, dx), lambda b: (b, 0)),
            pl.BlockSpec((dx, n), lambda b: (0, 0)),
            pl.BlockSpec((_BLK, dz), lambda b: (b, 0)),
            pl.BlockSpec((dz, n), lambda b: (0, 0)),
        ],
        out_specs=pl.BlockSpec((1, 1), lambda b: (0, 0)),
        out_shape=jax.ShapeDtypeStruct((1, 1), jnp.float32),
        scratch_shapes=[
            pltpu.VMEM((1, n), jnp.float32),
            pltpu.VMEM((1, n), jnp.float32),
            pltpu.VMEM((_BLK, n), jnp.float32),
        ],
    )(X, XT, Z, ZT)

    return (out[0, 0] / (n * _K * _K)).astype(jnp.float32)
